# Initial kernel scaffold; baseline (speedup 1.0000x reference)
import jax
import jax.numpy as jnp
from jax.experimental import pallas as pl


def kernel(user_emb, item_emb, W1_e, b1_e, W2_e, b2_e, W1_n, b1_n, W2_n, b2_n, eps_edge, eps_node, all_h_list, all_t_list):
    raise NotImplementedError("write your pallas kernel here")


# speedup vs baseline: 2.0087x; 2.0087x over previous
import functools

import jax
import jax.numpy as jnp
from jax import lax
from jax.experimental import pallas as pl
from jax.experimental.pallas import tpu as pltpu
from jax.experimental.pallas import tpu_sc as plsc

N_USERS = 6000
N_ITEMS = 4000
N = N_USERS + N_ITEMS
E = 320000
D = 128
L_LAYERS = 2
BIAS = 0.0001

NC, NS, LANE = 2, 16, 16
NW = NC * NS
CHUNK = 128
NCHUNK = 80
EPW = NCHUNK * CHUNK
EP = NW * EPW
NP = 10240
RPT = NP // NS
ZR = 128
BN = 1024
GRID_N = NP // BN

_SC_MESH = plsc.VectorSubcoreMesh(
    core_axis_name="c", subcore_axis_name="s", num_cores=NC, num_subcores=NS)


def _worker_id():
    return lax.axis_index("c") * NS + lax.axis_index("s")


def _dump_acc(acc_sh, out_hbm, cid, base):
    pltpu.sync_copy(acc_sh.at[pl.ds(base, RPT)], out_hbm.at[cid].at[pl.ds(base, RPT)])


@functools.partial(
    pl.kernel,
    out_type=jax.ShapeDtypeStruct((NC, NP, D), jnp.float32),
    mesh=_SC_MESH,
    compiler_params=pltpu.CompilerParams(needs_layout_passes=False),
    scratch_types=[
        pltpu.VMEM((NCHUNK, CHUNK), jnp.int32),
        pltpu.VMEM((CHUNK, D), jnp.float32),
        pltpu.VMEM_SHARED((NP, D), jnp.float32),
    ],
)
def _deg_kernel(h3, ones128, zeros128, out, h2_v, ones_v, acc_sh):
    cid = lax.axis_index("c")
    sid = lax.axis_index("s")
    wid = _worker_id()
    base = sid * RPT
    for j in range(RPT // ZR):
        pltpu.sync_copy(zeros128, acc_sh.at[pl.ds(base + j * ZR, ZR)])
    pltpu.sync_copy(ones128, ones_v)
    pltpu.sync_copy(h3.at[wid], h2_v)
    plsc.subcore_barrier()

    def chunk(k, carry):
        pltpu.sync_copy(ones_v, acc_sh.at[h2_v.at[k]], add=True)
        return carry

    lax.fori_loop(0, NCHUNK, chunk, 0)
    plsc.subcore_barrier()
    _dump_acc(acc_sh, out, cid, base)


@functools.partial(
    pl.kernel,
    out_type=jax.ShapeDtypeStruct((NW, EPW), jnp.float32),
    mesh=_SC_MESH,
    compiler_params=pltpu.CompilerParams(needs_layout_passes=False),
    scratch_types=[
        pltpu.VMEM((NCHUNK, CHUNK), jnp.int32),
        pltpu.VMEM((NCHUNK, CHUNK), jnp.int32),
        pltpu.VMEM((EPW,), jnp.float32),
        pltpu.VMEM((EPW,), jnp.float32),
        pltpu.VMEM((CHUNK, D), jnp.float32),
        pltpu.VMEM((CHUNK, D), jnp.float32),
        pltpu.VMEM((D, D), jnp.float32),
        pltpu.SemaphoreType.DMA,
    ],
)
def _edge_w_kernel(h3, t3, p1, p2, gum2, w2s,
                   w_out,
                   h2_v, t2_v, gum_v, w_v, r1_v, r2_v, w2s_v, sem):
    wid = _worker_id()
    pltpu.sync_copy(gum2.at[wid], gum_v)
    pltpu.sync_copy(w2s, w2s_v)
    pltpu.sync_copy(h3.at[wid], h2_v)
    pltpu.sync_copy(t3.at[wid], t2_v)

    def chunk(k, carry):
        cp1 = pltpu.async_copy(p1.at[h2_v.at[k]], r1_v, sem)
        cp2 = pltpu.async_copy(p2.at[t2_v.at[k]], r2_v, sem)
        cp1.wait()
        cp2.wait()

        def grp(g, c3):
            e16 = g * LANE + lax.iota(jnp.int32, LANE)
            accs = [jnp.zeros((LANE,), jnp.float32) for _ in range(4)]
            for d in range(D):
                col = jnp.full((LANE,), d, jnp.int32)
                v1 = plsc.load_gather(r1_v, [e16, col])
                v2 = plsc.load_gather(r2_v, [e16, col])
                w2d = w2s_v[d, pl.ds(0, LANE)]
                accs[d % 4] = accs[d % 4] + jnp.maximum(v1 + v2, 0.0) * w2d
            s16 = (accs[0] + accs[1]) + (accs[2] + accs[3])
            gm = gum_v[pl.ds(k * CHUNK + g * LANE, LANE)]
            w16 = 1.0 / (1.0 + jnp.exp(-(s16 + gm)))
            w_v[pl.ds(k * CHUNK + g * LANE, LANE)] = w16
            return c3

        lax.fori_loop(0, CHUNK // LANE, grp, 0)
        return carry

    lax.fori_loop(0, NCHUNK, chunk, 0)
    pltpu.sync_copy(w_v, w_out.at[wid])


@functools.partial(
    pl.kernel,
    out_type=(
        jax.ShapeDtypeStruct((NC, NP, D), jnp.float32),
        jax.ShapeDtypeStruct((NC, NP, D), jnp.float32),
        jax.ShapeDtypeStruct((NC, NP, D), jnp.float32),
        jax.ShapeDtypeStruct((NC, NP, D), jnp.float32),
    ),
    mesh=_SC_MESH,
    compiler_params=pltpu.CompilerParams(needs_layout_passes=False),
    scratch_types=[
        pltpu.VMEM((NCHUNK, CHUNK), jnp.int32),
        pltpu.VMEM((NCHUNK, CHUNK), jnp.int32),
        pltpu.VMEM((EPW,), jnp.float32),
        pltpu.VMEM((CHUNK, D), jnp.float32),
        pltpu.VMEM_SHARED((NP, D), jnp.float32),
        pltpu.SemaphoreType.DMA,
    ],
)
def _scatter4_kernel(h3, t3, s0, ne_s, e1, w2, zeros128,
                     acc0_out, acc2_out, acc1_out, rs_out,
                     h2_v, t2_v, w_v, r_v, acc_sh, sem):
    cid = lax.axis_index("c")
    sid = lax.axis_index("s")
    wid = _worker_id()
    base = sid * RPT

    def zero_acc():
        for j in range(RPT // ZR):
            pltpu.sync_copy(zeros128, acc_sh.at[pl.ds(base + j * ZR, ZR)])

    zero_acc()
    pltpu.sync_copy(h3.at[wid], h2_v)
    pltpu.sync_copy(t3.at[wid], t2_v)
    pltpu.sync_copy(w2.at[wid], w_v)
    plsc.subcore_barrier()

    def stream_phase(table):
        def chunk(k, carry):
            pltpu.async_copy(table.at[t2_v.at[k]], r_v, sem).wait()
            pltpu.sync_copy(r_v, acc_sh.at[h2_v.at[k]], add=True)
            return carry

        lax.fori_loop(0, NCHUNK, chunk, 0)

    def next_phase(out_hbm):
        plsc.subcore_barrier()
        _dump_acc(acc_sh, out_hbm, cid, base)
        zero_acc()
        plsc.subcore_barrier()

    stream_phase(s0)
    next_phase(acc0_out)

    stream_phase(ne_s)
    next_phase(acc2_out)

    def chunk_c(k, carry):
        pltpu.async_copy(e1.at[t2_v.at[k]], r_v, sem).wait()

        def edge(e, c2):
            ws = plsc.load_gather(w_v, [jnp.full((LANE,), k * CHUNK + e, jnp.int32)])
            for c in range(D // LANE):
                r_v[e, pl.ds(c * LANE, LANE)] = r_v[e, pl.ds(c * LANE, LANE)] * ws
            return c2

        lax.fori_loop(0, CHUNK, edge, 0)
        pltpu.sync_copy(r_v, acc_sh.at[h2_v.at[k]], add=True)
        return carry

    lax.fori_loop(0, NCHUNK, chunk_c, 0)
    next_phase(acc1_out)

    pltpu.sync_copy(zeros128, r_v)

    def chunk_d(k, carry):
        def edge(e, c2):
            ws = plsc.load_gather(w_v, [jnp.full((LANE,), k * CHUNK + e, jnp.int32)])
            r_v[e, pl.ds(0, LANE)] = ws
            return c2

        lax.fori_loop(0, CHUNK, edge, 0)
        pltpu.sync_copy(r_v, acc_sh.at[h2_v.at[k]], add=True)
        return carry

    lax.fori_loop(0, NCHUNK, chunk_d, 0)
    plsc.subcore_barrier()
    _dump_acc(acc_sh, rs_out, cid, base)


def _prep_n_body(degp0, degp1, e0, dis, s0):
    deg = degp0[:, 0] + degp1[:, 0]
    d = jnp.where(deg > 0, lax.rsqrt(jnp.maximum(deg, 1e-30)), 0.0)
    dis[...] = d[:, None]
    s0[...] = d[:, None] * e0[...]


def _prep_n(degp0, degp1, e0):
    row = pl.BlockSpec((BN, D), lambda i: (i, 0))
    return pl.pallas_call(
        _prep_n_body,
        grid=(GRID_N,),
        in_specs=[row, row, row],
        out_specs=[pl.BlockSpec((BN, 1), lambda i: (i, 0)), row],
        out_shape=[
            jax.ShapeDtypeStruct((NP, 1), jnp.float32),
            jax.ShapeDtypeStruct((NP, D), jnp.float32),
        ],
    )(degp0, degp1, e0)


_EW = 128
_ER = E // _EW


def _prep_e_body(eps, b2, gum):
    lin = (2.0 * BIAS - 1.0) * eps[...] + (1.0 - BIAS)
    gum[...] = -jnp.log(-jnp.log(lin)) + b2[0, 0]


def _prep_e(eps_layer, b2_layer):
    out = pl.pallas_call(
        _prep_e_body,
        grid=(1,),
        in_specs=[
            pl.BlockSpec((_ER, _EW), lambda i: (0, 0)),
            pl.BlockSpec((1, 1), lambda i: (0, 0)),
        ],
        out_specs=pl.BlockSpec((_ER, _EW), lambda i: (0, 0)),
        out_shape=jax.ShapeDtypeStruct((_ER, _EW), jnp.float32),
    )(eps_layer.reshape(_ER, _EW), b2_layer.reshape(1, 1))
    return out.reshape(E)


def _dense_a_body(e1, e2, epsn, dis, w1a, w1b, b1e, w1n, b1n, w2n, b2n,
                  p1, p2, ne_s):
    p1[...] = jnp.dot(e1[...], w1a[...], preferred_element_type=jnp.float32) + b1e[...]
    p2[...] = jnp.dot(e1[...], w1b[...], preferred_element_type=jnp.float32)
    hid = jnp.maximum(jnp.dot(e2[...], w1n[...], preferred_element_type=jnp.float32) + b1n[...], 0.0)
    lg = jnp.dot(hid, w2n[...], preferred_element_type=jnp.float32) + b2n[...]
    lin = (2.0 * BIAS - 1.0) * epsn[...] + (1.0 - BIAS)
    gate = jax.nn.sigmoid(-jnp.log(-jnp.log(lin)) + lg)
    ne_s[...] = dis[...] * gate * e2[...]


def _dense_a(e1t, e2t, epsn, dis, w1a, w1b, b1e, w1n, b1n, w2n, b2n):
    row = pl.BlockSpec((BN, D), lambda i: (i, 0))
    mat = pl.BlockSpec((D, D), lambda i: (0, 0))
    vec = pl.BlockSpec((1, D), lambda i: (0, 0))
    return pl.pallas_call(
        _dense_a_body,
        grid=(GRID_N,),
        in_specs=[row, row, row, pl.BlockSpec((BN, 1), lambda i: (i, 0)),
                  mat, mat, vec, mat, vec, mat, vec],
        out_specs=[row, row, row],
        out_shape=[jax.ShapeDtypeStruct((NP, D), jnp.float32)] * 3,
    )(e1t, e2t, epsn, dis, w1a, w1b, b1e, w1n, b1n, w2n, b2n)


def _combine_body(e0, e1, e2, a00, a01, a10, a11, a20, a21, rs0, rs1, dis,
                  s0i, s1i, s2i,
                  e0n, e1n, e2n, s0n, s0o, s1o, s2o):
    rs = rs0[:, 0] + rs1[:, 0]
    inv = jnp.where(rs > 0, 1.0 / jnp.maximum(rs, 1e-30), 0.0)[:, None]
    d = dis[...]
    v0 = e0[...] + d * (a00[...] + a01[...])
    v1 = e1[...] + inv * (a10[...] + a11[...])
    v2 = e2[...] + d * (a20[...] + a21[...])
    e0n[...] = v0
    e1n[...] = v1
    e2n[...] = v2
    s0n[...] = d * v0
    s0o[...] = s0i[...] + v0
    s1o[...] = s1i[...] + v1
    s2o[...] = s2i[...] + v2


def _combine(e0t, e1t, e2t, a0p, a1p, a2p, rsp, dis, sums):
    row = pl.BlockSpec((BN, D), lambda i: (i, 0))
    return pl.pallas_call(
        _combine_body,
        grid=(GRID_N,),
        in_specs=[row, row, row, row, row, row, row, row, row,
                  row, row, pl.BlockSpec((BN, 1), lambda i: (i, 0)),
                  row, row, row],
        out_specs=[row] * 7,
        out_shape=[jax.ShapeDtypeStruct((NP, D), jnp.float32)] * 7,
    )(e0t, e1t, e2t, a0p[0], a0p[1], a1p[0], a1p[1], a2p[0], a2p[1],
      rsp[0], rsp[1], dis, sums[0], sums[1], sums[2])


def kernel(user_emb, item_emb, W1_e, b1_e, W2_e, b2_e, W1_n, b1_n, W2_n, b2_n,
           eps_edge, eps_node, all_h_list, all_t_list):
    e0 = jnp.pad(jnp.concatenate([user_emb, item_emb], axis=0),
                 ((0, NP - N), (0, 0)))
    h3 = jnp.pad(all_h_list.astype(jnp.int32), (0, EP - E),
                 constant_values=N).reshape(NW, NCHUNK, CHUNK)
    t3 = jnp.pad(all_t_list.astype(jnp.int32), (0, EP - E),
                 constant_values=N).reshape(NW, NCHUNK, CHUNK)
    ones128 = jnp.ones((CHUNK, D), jnp.float32)
    zeros128 = jnp.zeros((ZR, D), jnp.float32)

    degp = _deg_kernel(h3, ones128, zeros128)
    dis, s0t = _prep_n(degp[0], degp[1], e0)
    gum = jnp.stack([_prep_e(eps_edge[i, :, 0], b2_e[i]) for i in range(L_LAYERS)])

    e0t, e1t, e2t = e0, e0, e0
    sums = (e0, e0, e0)
    for i in range(L_LAYERS):
        epsn = jnp.pad(eps_node[i], ((0, NP - N), (0, 0)), constant_values=0.5)
        p1, p2, ne_s = _dense_a(
            e1t, e2t, epsn, dis,
            W1_e[i, :D, :], W1_e[i, D:, :], b1_e[i].reshape(1, D),
            W1_n[i], b1_n[i].reshape(1, D), W2_n[i], b2_n[i].reshape(1, D))
        gum_i = jnp.pad(gum[i], (0, EP - E)).reshape(NW, EPW)
        w2s = jnp.broadcast_to(W2_e[i, :, :1], (D, D))
        w_out = _edge_w_kernel(h3, t3, p1, p2, gum_i, w2s)
        acc0p, acc2p, acc1p, rsp = _scatter4_kernel(
            h3, t3, s0t, ne_s, e1t, w_out, zeros128)
        e0t, e1t, e2t, s0t, su0, su1, su2 = _combine(
            e0t, e1t, e2t, acc0p, acc1p, acc2p, rsp, dis, sums)
        sums = (su0, su1, su2)

    return jnp.stack(sums, axis=0)[:, :N, :]


# speedup vs baseline: 2.5295x; 1.2593x over previous
import functools

import jax
import jax.numpy as jnp
from jax import lax
from jax.experimental import pallas as pl
from jax.experimental.pallas import tpu as pltpu
from jax.experimental.pallas import tpu_sc as plsc

N_USERS = 6000
N_ITEMS = 4000
N = N_USERS + N_ITEMS
E = 320000
D = 128
L_LAYERS = 2
BIAS = 0.0001

NC, NS, LANE = 2, 16, 16
NW = NC * NS
CHUNK = 128
NCHUNK = 80
EPW = NCHUNK * CHUNK
EP = NW * EPW
NP = 10240
RPT = NP // NS
ZR = 128
BN = 1024
GRID_N = NP // BN

_SC_MESH = plsc.VectorSubcoreMesh(
    core_axis_name="c", subcore_axis_name="s", num_cores=NC, num_subcores=NS)


def _worker_id():
    return lax.axis_index("c") * NS + lax.axis_index("s")


def _dump_acc(acc_sh, out_hbm, cid, base):
    pltpu.sync_copy(acc_sh.at[pl.ds(base, RPT)], out_hbm.at[cid].at[pl.ds(base, RPT)])


@functools.partial(
    pl.kernel,
    out_type=jax.ShapeDtypeStruct((NC, NP, D), jnp.float32),
    mesh=_SC_MESH,
    compiler_params=pltpu.CompilerParams(needs_layout_passes=False),
    scratch_types=[
        pltpu.VMEM((NCHUNK, CHUNK), jnp.int32),
        pltpu.VMEM((CHUNK, D), jnp.float32),
        pltpu.VMEM_SHARED((NP, D), jnp.float32),
    ],
)
def _deg_kernel(h3, ones128, zeros128, out, h2_v, ones_v, acc_sh):
    cid = lax.axis_index("c")
    sid = lax.axis_index("s")
    wid = _worker_id()
    base = sid * RPT
    for j in range(RPT // ZR):
        pltpu.sync_copy(zeros128, acc_sh.at[pl.ds(base + j * ZR, ZR)])
    pltpu.sync_copy(ones128, ones_v)
    pltpu.sync_copy(h3.at[wid], h2_v)
    plsc.subcore_barrier()

    def chunk(k, carry):
        pltpu.sync_copy(ones_v, acc_sh.at[h2_v.at[k]], add=True)
        return carry

    lax.fori_loop(0, NCHUNK, chunk, 0)
    plsc.subcore_barrier()
    _dump_acc(acc_sh, out, cid, base)


@functools.partial(
    pl.kernel,
    out_type=jax.ShapeDtypeStruct((NW, EPW), jnp.float32),
    mesh=_SC_MESH,
    compiler_params=pltpu.CompilerParams(needs_layout_passes=False),
    scratch_types=[
        pltpu.VMEM((NCHUNK, CHUNK), jnp.int32),
        pltpu.VMEM((NCHUNK, CHUNK), jnp.int32),
        pltpu.VMEM((EPW,), jnp.float32),
        pltpu.VMEM((EPW,), jnp.float32),
        pltpu.VMEM((CHUNK, D), jnp.float32),
        pltpu.VMEM((CHUNK, D), jnp.float32),
        pltpu.VMEM((CHUNK, D), jnp.float32),
        pltpu.VMEM((CHUNK, D), jnp.float32),
        pltpu.VMEM((D, D), jnp.float32),
        pltpu.SemaphoreType.DMA,
    ],
)
def _edge_w_kernel(h3, t3, p1, p2, gum2, w2s,
                   w_out,
                   h2_v, t2_v, gum_v, w_v, r1a_v, r2a_v, r1b_v, r2b_v,
                   w2s_v, sem):
    wid = _worker_id()
    pltpu.sync_copy(gum2.at[wid], gum_v)
    pltpu.sync_copy(w2s, w2s_v)
    pltpu.sync_copy(h3.at[wid], h2_v)
    pltpu.sync_copy(t3.at[wid], t2_v)

    def issue(k, r1_v, r2_v):
        pltpu.async_copy(p1.at[h2_v.at[k]], r1_v, sem)
        pltpu.async_copy(p2.at[t2_v.at[k]], r2_v, sem)

    def wait(r1_v, r2_v):
        pltpu.make_async_copy(p1.at[h2_v.at[0]], r1_v, sem).wait()
        pltpu.make_async_copy(p2.at[t2_v.at[0]], r2_v, sem).wait()

    def compute(k, r1_v, r2_v):
        def grp(g, c3):
            e16 = g * LANE + lax.iota(jnp.int32, LANE)
            accs = [jnp.zeros((LANE,), jnp.float32) for _ in range(4)]
            for d in range(D):
                col = jnp.full((LANE,), d, jnp.int32)
                v1 = plsc.load_gather(r1_v, [e16, col])
                v2 = plsc.load_gather(r2_v, [e16, col])
                w2d = w2s_v[d, pl.ds(0, LANE)]
                accs[d % 4] = accs[d % 4] + jnp.maximum(v1 + v2, 0.0) * w2d
            s16 = (accs[0] + accs[1]) + (accs[2] + accs[3])
            gm = gum_v[pl.ds(k * CHUNK + g * LANE, LANE)]
            w16 = 1.0 / (1.0 + jnp.exp(-(s16 + gm)))
            w_v[pl.ds(k * CHUNK + g * LANE, LANE)] = w16
            return c3

        lax.fori_loop(0, CHUNK // LANE, grp, 0)

    issue(0, r1a_v, r2a_v)

    def chunk2(k2, carry):
        k = 2 * k2
        wait(r1a_v, r2a_v)
        issue(k + 1, r1b_v, r2b_v)
        compute(k, r1a_v, r2a_v)
        wait(r1b_v, r2b_v)
        issue(jnp.minimum(k + 2, NCHUNK - 1), r1a_v, r2a_v)
        compute(k + 1, r1b_v, r2b_v)
        return carry

    lax.fori_loop(0, NCHUNK // 2, chunk2, 0)
    wait(r1a_v, r2a_v)
    pltpu.sync_copy(w_v, w_out.at[wid])


@functools.partial(
    pl.kernel,
    out_type=(
        jax.ShapeDtypeStruct((NC, NP, D), jnp.float32),
        jax.ShapeDtypeStruct((NC, NP, D), jnp.float32),
        jax.ShapeDtypeStruct((NC, NP, D), jnp.float32),
        jax.ShapeDtypeStruct((NC, NP, D), jnp.float32),
    ),
    mesh=_SC_MESH,
    compiler_params=pltpu.CompilerParams(needs_layout_passes=False),
    scratch_types=[
        pltpu.VMEM((1, CHUNK), jnp.int32),
        pltpu.VMEM((1, CHUNK), jnp.int32),
        pltpu.VMEM((1, CHUNK), jnp.float32),
        pltpu.VMEM((1, CHUNK), jnp.int32),
        pltpu.VMEM((1, CHUNK), jnp.int32),
        pltpu.VMEM((1, CHUNK), jnp.float32),
        pltpu.VMEM((CHUNK, D), jnp.float32),
        pltpu.VMEM((CHUNK, D), jnp.float32),
        pltpu.VMEM_SHARED((NP, D), jnp.float32),
        pltpu.SemaphoreType.DMA,
    ],
)
def _scatter4_kernel(h4, t4, s0, ne_s, e1, w4, zeros128,
                     acc0_out, acc2_out, acc1_out, rs_out,
                     hqa_v, tqa_v, wqa_v, hqb_v, tqb_v, wqb_v,
                     ra_v, rb_v, acc_sh, sem):
    cid = lax.axis_index("c")
    sid = lax.axis_index("s")
    wid = _worker_id()
    base = sid * RPT
    zeros16i = jnp.zeros((LANE,), jnp.int32)

    def zero_acc():
        for j in range(RPT // ZR):
            pltpu.sync_copy(zeros128, acc_sh.at[pl.ds(base + j * ZR, ZR)])

    zero_acc()
    plsc.subcore_barrier()

    def load_idx(k, hq, tq, wq, need_t, need_w):
        pltpu.sync_copy(h4.at[wid, k], hq)
        if need_t:
            pltpu.sync_copy(t4.at[wid, k], tq)
        if need_w:
            pltpu.sync_copy(w4.at[wid, k], wq)

    def scale(buf, wq):
        def edge(e, c2):
            ws = plsc.load_gather(wq, [zeros16i, jnp.full((LANE,), e, jnp.int32)])
            for c in range(D // LANE):
                buf[e, pl.ds(c * LANE, LANE)] = buf[e, pl.ds(c * LANE, LANE)] * ws
            return c2

        lax.fori_loop(0, CHUNK, edge, 0)

    def stream_phase(table, scale_w):
        def gissue(tq, buf):
            pltpu.async_copy(table.at[tq.at[0]], buf, sem)

        def gwait(buf):
            pltpu.make_async_copy(table.at[tqa_v.at[0]], buf, sem).wait()

        load_idx(0, hqa_v, tqa_v, wqa_v, True, scale_w)
        gissue(tqa_v, ra_v)

        def chunk2(k2, carry):
            k = 2 * k2
            load_idx(k + 1, hqb_v, tqb_v, wqb_v, True, scale_w)
            gissue(tqb_v, rb_v)
            gwait(ra_v)
            if scale_w:
                scale(ra_v, wqa_v)
            pltpu.sync_copy(ra_v, acc_sh.at[hqa_v.at[0]], add=True)
            load_idx(jnp.minimum(k + 2, NCHUNK - 1), hqa_v, tqa_v, wqa_v,
                     True, scale_w)
            gissue(tqa_v, ra_v)
            gwait(rb_v)
            if scale_w:
                scale(rb_v, wqb_v)
            pltpu.sync_copy(rb_v, acc_sh.at[hqb_v.at[0]], add=True)
            return carry

        lax.fori_loop(0, NCHUNK // 2, chunk2, 0)
        gwait(ra_v)

    def next_phase(out_hbm):
        plsc.subcore_barrier()
        _dump_acc(acc_sh, out_hbm, cid, base)
        zero_acc()
        plsc.subcore_barrier()

    stream_phase(s0, scale_w=False)
    next_phase(acc0_out)

    stream_phase(ne_s, scale_w=False)
    next_phase(acc2_out)

    stream_phase(e1, scale_w=True)
    next_phase(acc1_out)

    pltpu.sync_copy(zeros128, ra_v)

    def chunk_d(k, carry):
        load_idx(k, hqa_v, tqa_v, wqa_v, False, True)

        def edge(e, c2):
            ws = plsc.load_gather(wqa_v, [zeros16i, jnp.full((LANE,), e, jnp.int32)])
            ra_v[e, pl.ds(0, LANE)] = ws
            return c2

        lax.fori_loop(0, CHUNK, edge, 0)
        pltpu.sync_copy(ra_v, acc_sh.at[hqa_v.at[0]], add=True)
        return carry

    lax.fori_loop(0, NCHUNK, chunk_d, 0)
    plsc.subcore_barrier()
    _dump_acc(acc_sh, rs_out, cid, base)


def _prep_n_body(degp0, degp1, e0, dis, s0):
    deg = degp0[:, 0] + degp1[:, 0]
    d = jnp.where(deg > 0, lax.rsqrt(jnp.maximum(deg, 1e-30)), 0.0)
    dis[...] = d[:, None]
    s0[...] = d[:, None] * e0[...]


def _prep_n(degp0, degp1, e0):
    row = pl.BlockSpec((BN, D), lambda i: (i, 0))
    return pl.pallas_call(
        _prep_n_body,
        grid=(GRID_N,),
        in_specs=[row, row, row],
        out_specs=[pl.BlockSpec((BN, 1), lambda i: (i, 0)), row],
        out_shape=[
            jax.ShapeDtypeStruct((NP, 1), jnp.float32),
            jax.ShapeDtypeStruct((NP, D), jnp.float32),
        ],
    )(degp0, degp1, e0)


_EW = 128
_ER = E // _EW


def _prep_e_body(eps, b2, gum):
    lin = (2.0 * BIAS - 1.0) * eps[...] + (1.0 - BIAS)
    gum[...] = -jnp.log(-jnp.log(lin)) + b2[0, 0]


def _prep_e(eps_layer, b2_layer):
    out = pl.pallas_call(
        _prep_e_body,
        grid=(1,),
        in_specs=[
            pl.BlockSpec((_ER, _EW), lambda i: (0, 0)),
            pl.BlockSpec((1, 1), lambda i: (0, 0)),
        ],
        out_specs=pl.BlockSpec((_ER, _EW), lambda i: (0, 0)),
        out_shape=jax.ShapeDtypeStruct((_ER, _EW), jnp.float32),
    )(eps_layer.reshape(_ER, _EW), b2_layer.reshape(1, 1))
    return out.reshape(E)


def _dense_a_body(e1, e2, epsn, dis, w1a, w1b, b1e, w1n, b1n, w2n, b2n,
                  p1, p2, ne_s):
    p1[...] = jnp.dot(e1[...], w1a[...], preferred_element_type=jnp.float32) + b1e[...]
    p2[...] = jnp.dot(e1[...], w1b[...], preferred_element_type=jnp.float32)
    hid = jnp.maximum(jnp.dot(e2[...], w1n[...], preferred_element_type=jnp.float32) + b1n[...], 0.0)
    lg = jnp.dot(hid, w2n[...], preferred_element_type=jnp.float32) + b2n[...]
    lin = (2.0 * BIAS - 1.0) * epsn[...] + (1.0 - BIAS)
    gate = jax.nn.sigmoid(-jnp.log(-jnp.log(lin)) + lg)
    ne_s[...] = dis[...] * gate * e2[...]


def _dense_a(e1t, e2t, epsn, dis, w1a, w1b, b1e, w1n, b1n, w2n, b2n):
    row = pl.BlockSpec((BN, D), lambda i: (i, 0))
    mat = pl.BlockSpec((D, D), lambda i: (0, 0))
    vec = pl.BlockSpec((1, D), lambda i: (0, 0))
    return pl.pallas_call(
        _dense_a_body,
        grid=(GRID_N,),
        in_specs=[row, row, row, pl.BlockSpec((BN, 1), lambda i: (i, 0)),
                  mat, mat, vec, mat, vec, mat, vec],
        out_specs=[row, row, row],
        out_shape=[jax.ShapeDtypeStruct((NP, D), jnp.float32)] * 3,
    )(e1t, e2t, epsn, dis, w1a, w1b, b1e, w1n, b1n, w2n, b2n)


def _combine_body(e0, e1, e2, a00, a01, a10, a11, a20, a21, rs0, rs1, dis,
                  s0i, s1i, s2i,
                  e0n, e1n, e2n, s0n, s0o, s1o, s2o):
    rs = rs0[:, 0] + rs1[:, 0]
    inv = jnp.where(rs > 0, 1.0 / jnp.maximum(rs, 1e-30), 0.0)[:, None]
    d = dis[...]
    v0 = e0[...] + d * (a00[...] + a01[...])
    v1 = e1[...] + inv * (a10[...] + a11[...])
    v2 = e2[...] + d * (a20[...] + a21[...])
    e0n[...] = v0
    e1n[...] = v1
    e2n[...] = v2
    s0n[...] = d * v0
    s0o[...] = s0i[...] + v0
    s1o[...] = s1i[...] + v1
    s2o[...] = s2i[...] + v2


def _combine(e0t, e1t, e2t, a0p, a1p, a2p, rsp, dis, sums):
    row = pl.BlockSpec((BN, D), lambda i: (i, 0))
    return pl.pallas_call(
        _combine_body,
        grid=(GRID_N,),
        in_specs=[row, row, row, row, row, row, row, row, row,
                  row, row, pl.BlockSpec((BN, 1), lambda i: (i, 0)),
                  row, row, row],
        out_specs=[row] * 7,
        out_shape=[jax.ShapeDtypeStruct((NP, D), jnp.float32)] * 7,
    )(e0t, e1t, e2t, a0p[0], a0p[1], a1p[0], a1p[1], a2p[0], a2p[1],
      rsp[0], rsp[1], dis, sums[0], sums[1], sums[2])


def kernel(user_emb, item_emb, W1_e, b1_e, W2_e, b2_e, W1_n, b1_n, W2_n, b2_n,
           eps_edge, eps_node, all_h_list, all_t_list):
    e0 = jnp.pad(jnp.concatenate([user_emb, item_emb], axis=0),
                 ((0, NP - N), (0, 0)))
    h3 = jnp.pad(all_h_list.astype(jnp.int32), (0, EP - E),
                 constant_values=N).reshape(NW, NCHUNK, CHUNK)
    t3 = jnp.pad(all_t_list.astype(jnp.int32), (0, EP - E),
                 constant_values=N).reshape(NW, NCHUNK, CHUNK)
    ones128 = jnp.ones((CHUNK, D), jnp.float32)
    zeros128 = jnp.zeros((ZR, D), jnp.float32)

    degp = _deg_kernel(h3, ones128, zeros128)
    dis, s0t = _prep_n(degp[0], degp[1], e0)
    gum = jnp.stack([_prep_e(eps_edge[i, :, 0], b2_e[i]) for i in range(L_LAYERS)])

    e0t, e1t, e2t = e0, e0, e0
    sums = (e0, e0, e0)
    for i in range(L_LAYERS):
        epsn = jnp.pad(eps_node[i], ((0, NP - N), (0, 0)), constant_values=0.5)
        p1, p2, ne_s = _dense_a(
            e1t, e2t, epsn, dis,
            W1_e[i, :D, :], W1_e[i, D:, :], b1_e[i].reshape(1, D),
            W1_n[i], b1_n[i].reshape(1, D), W2_n[i], b2_n[i].reshape(1, D))
        gum_i = jnp.pad(gum[i], (0, EP - E)).reshape(NW, EPW)
        w2s = jnp.broadcast_to(W2_e[i, :, :1], (D, D))
        w_out = _edge_w_kernel(h3, t3, p1, p2, gum_i, w2s)
        acc0p, acc2p, acc1p, rsp = _scatter4_kernel(
            h3.reshape(NW, NCHUNK, 1, CHUNK), t3.reshape(NW, NCHUNK, 1, CHUNK),
            s0t, ne_s, e1t, w_out.reshape(NW, NCHUNK, 1, CHUNK), zeros128)
        e0t, e1t, e2t, s0t, su0, su1, su2 = _combine(
            e0t, e1t, e2t, acc0p, acc1p, acc2p, rsp, dis, sums)
        sums = (su0, su1, su2)

    return jnp.stack(sums, axis=0)[:, :N, :]


# speedup vs baseline: 2.5646x; 1.0139x over previous
import functools

import jax
import jax.numpy as jnp
from jax import lax
from jax.experimental import pallas as pl
from jax.experimental.pallas import tpu as pltpu
from jax.experimental.pallas import tpu_sc as plsc

N_USERS = 6000
N_ITEMS = 4000
N = N_USERS + N_ITEMS
E = 320000
D = 128
L_LAYERS = 2
BIAS = 0.0001

NC, NS, LANE = 2, 16, 16
NW = NC * NS
CHUNK = 128
NCHUNK = 80
EPW = NCHUNK * CHUNK
EP = NW * EPW
NP = 10240
RPT = NP // NS
ZR = 128
BN = 1024
GRID_N = NP // BN

_SC_MESH = plsc.VectorSubcoreMesh(
    core_axis_name="c", subcore_axis_name="s", num_cores=NC, num_subcores=NS)


def _worker_id():
    return lax.axis_index("c") * NS + lax.axis_index("s")


def _dump_acc(acc_sh, out_hbm, cid, base):
    pltpu.sync_copy(acc_sh.at[pl.ds(base, RPT)], out_hbm.at[cid].at[pl.ds(base, RPT)])


@functools.partial(
    pl.kernel,
    out_type=jax.ShapeDtypeStruct((NC, NP, D), jnp.float32),
    mesh=_SC_MESH,
    compiler_params=pltpu.CompilerParams(needs_layout_passes=False),
    scratch_types=[
        pltpu.VMEM((NCHUNK, CHUNK), jnp.int32),
        pltpu.VMEM((CHUNK, D), jnp.float32),
        pltpu.VMEM_SHARED((NP, D), jnp.float32),
    ],
)
def _deg_kernel(h3, ones128, zeros128, out, h2_v, ones_v, acc_sh):
    cid = lax.axis_index("c")
    sid = lax.axis_index("s")
    wid = _worker_id()
    base = sid * RPT
    for j in range(RPT // ZR):
        pltpu.sync_copy(zeros128, acc_sh.at[pl.ds(base + j * ZR, ZR)])
    pltpu.sync_copy(ones128, ones_v)
    pltpu.sync_copy(h3.at[wid], h2_v)
    plsc.subcore_barrier()

    def chunk(k, carry):
        pltpu.sync_copy(ones_v, acc_sh.at[h2_v.at[k]], add=True)
        return carry

    lax.fori_loop(0, NCHUNK, chunk, 0)
    plsc.subcore_barrier()
    _dump_acc(acc_sh, out, cid, base)


@functools.partial(
    pl.kernel,
    out_type=jax.ShapeDtypeStruct((NW, EPW), jnp.float32),
    mesh=_SC_MESH,
    compiler_params=pltpu.CompilerParams(needs_layout_passes=False),
    scratch_types=[
        pltpu.VMEM((NCHUNK, CHUNK), jnp.int32),
        pltpu.VMEM((NCHUNK, CHUNK), jnp.int32),
        pltpu.VMEM((EPW,), jnp.float32),
        pltpu.VMEM((EPW,), jnp.float32),
        pltpu.VMEM((CHUNK, D), jnp.float32),
        pltpu.VMEM((CHUNK, D), jnp.float32),
        pltpu.VMEM((CHUNK, D), jnp.float32),
        pltpu.VMEM((CHUNK, D), jnp.float32),
        pltpu.VMEM((D, D), jnp.float32),
        pltpu.SemaphoreType.DMA,
    ],
)
def _edge_w_kernel(h3, t3, p1, p2, gum2, w2s,
                   w_out,
                   h2_v, t2_v, gum_v, w_v, r1a_v, r2a_v, r1b_v, r2b_v,
                   w2s_v, sem):
    wid = _worker_id()
    pltpu.sync_copy(gum2.at[wid], gum_v)
    pltpu.sync_copy(w2s, w2s_v)
    pltpu.sync_copy(h3.at[wid], h2_v)
    pltpu.sync_copy(t3.at[wid], t2_v)

    def issue(k, r1_v, r2_v):
        pltpu.async_copy(p1.at[h2_v.at[k]], r1_v, sem)
        pltpu.async_copy(p2.at[t2_v.at[k]], r2_v, sem)

    def wait(r1_v, r2_v):
        pltpu.make_async_copy(p1.at[h2_v.at[0]], r1_v, sem).wait()
        pltpu.make_async_copy(p2.at[t2_v.at[0]], r2_v, sem).wait()

    def compute(k, r1_v, r2_v):
        def grp(g, c3):
            e16 = g * LANE + lax.iota(jnp.int32, LANE)
            accs = [jnp.zeros((LANE,), jnp.float32) for _ in range(4)]
            for d in range(D):
                col = jnp.full((LANE,), d, jnp.int32)
                v1 = plsc.load_gather(r1_v, [e16, col])
                v2 = plsc.load_gather(r2_v, [e16, col])
                w2d = w2s_v[d, pl.ds(0, LANE)]
                accs[d % 4] = accs[d % 4] + jnp.maximum(v1 + v2, 0.0) * w2d
            s16 = (accs[0] + accs[1]) + (accs[2] + accs[3])
            gm = gum_v[pl.ds(k * CHUNK + g * LANE, LANE)]
            w16 = 1.0 / (1.0 + jnp.exp(-(s16 + gm)))
            w_v[pl.ds(k * CHUNK + g * LANE, LANE)] = w16
            return c3

        lax.fori_loop(0, CHUNK // LANE, grp, 0)

    issue(0, r1a_v, r2a_v)

    def chunk2(k2, carry):
        k = 2 * k2
        wait(r1a_v, r2a_v)
        issue(k + 1, r1b_v, r2b_v)
        compute(k, r1a_v, r2a_v)
        wait(r1b_v, r2b_v)
        issue(jnp.minimum(k + 2, NCHUNK - 1), r1a_v, r2a_v)
        compute(k + 1, r1b_v, r2b_v)
        return carry

    lax.fori_loop(0, NCHUNK // 2, chunk2, 0)
    wait(r1a_v, r2a_v)
    pltpu.sync_copy(w_v, w_out.at[wid])


@functools.partial(
    pl.kernel,
    out_type=(
        jax.ShapeDtypeStruct((NC, NP, D), jnp.float32),
        jax.ShapeDtypeStruct((NC, NP, D), jnp.float32),
        jax.ShapeDtypeStruct((NC, NP, D), jnp.float32),
        jax.ShapeDtypeStruct((NC, NP, D), jnp.float32),
    ),
    mesh=_SC_MESH,
    compiler_params=pltpu.CompilerParams(needs_layout_passes=False),
    scratch_types=[
        pltpu.VMEM((1, CHUNK), jnp.int32),
        pltpu.VMEM((1, CHUNK), jnp.int32),
        pltpu.VMEM((1, CHUNK), jnp.float32),
        pltpu.VMEM((1, CHUNK), jnp.int32),
        pltpu.VMEM((1, CHUNK), jnp.int32),
        pltpu.VMEM((1, CHUNK), jnp.float32),
        pltpu.VMEM((CHUNK, D), jnp.float32),
        pltpu.VMEM((CHUNK, D), jnp.float32),
        pltpu.VMEM_SHARED((NP, D), jnp.float32),
        pltpu.SemaphoreType.DMA,
    ],
)
def _scatter4_kernel(h4, t4, s0, ne_s, e1, w4, zeros128,
                     acc0_out, acc2_out, acc1_out, rs_out,
                     hqa_v, tqa_v, wqa_v, hqb_v, tqb_v, wqb_v,
                     ra_v, rb_v, acc_sh, sem):
    cid = lax.axis_index("c")
    sid = lax.axis_index("s")
    wid = _worker_id()
    base = sid * RPT
    zeros16i = jnp.zeros((LANE,), jnp.int32)

    def zero_acc():
        for j in range(RPT // ZR):
            pltpu.sync_copy(zeros128, acc_sh.at[pl.ds(base + j * ZR, ZR)])

    zero_acc()
    plsc.subcore_barrier()

    def load_idx(k, hq, tq, wq, need_t, need_w):
        pltpu.sync_copy(h4.at[wid, k], hq)
        if need_t:
            pltpu.sync_copy(t4.at[wid, k], tq)
        if need_w:
            pltpu.sync_copy(w4.at[wid, k], wq)

    def scale(buf, wq):
        @plsc.parallel_loop(0, CHUNK, 1, unroll=4)
        def edge(e):
            ws = plsc.load_gather(wq, [zeros16i, jnp.full((LANE,), e, jnp.int32)])
            for c in range(D // LANE):
                buf[e, pl.ds(c * LANE, LANE)] = buf[e, pl.ds(c * LANE, LANE)] * ws

    def stream_phase(table, scale_w):
        def gissue(tq, buf):
            pltpu.async_copy(table.at[tq.at[0]], buf, sem)

        def gwait(buf):
            pltpu.make_async_copy(table.at[tqa_v.at[0]], buf, sem).wait()

        load_idx(0, hqa_v, tqa_v, wqa_v, True, scale_w)
        gissue(tqa_v, ra_v)

        def chunk2(k2, carry):
            k = 2 * k2
            load_idx(k + 1, hqb_v, tqb_v, wqb_v, True, scale_w)
            gissue(tqb_v, rb_v)
            gwait(ra_v)
            if scale_w:
                scale(ra_v, wqa_v)
            pltpu.sync_copy(ra_v, acc_sh.at[hqa_v.at[0]], add=True)
            load_idx(jnp.minimum(k + 2, NCHUNK - 1), hqa_v, tqa_v, wqa_v,
                     True, scale_w)
            gissue(tqa_v, ra_v)
            gwait(rb_v)
            if scale_w:
                scale(rb_v, wqb_v)
            pltpu.sync_copy(rb_v, acc_sh.at[hqb_v.at[0]], add=True)
            return carry

        lax.fori_loop(0, NCHUNK // 2, chunk2, 0)
        gwait(ra_v)

    def next_phase(out_hbm):
        plsc.subcore_barrier()
        _dump_acc(acc_sh, out_hbm, cid, base)
        zero_acc()
        plsc.subcore_barrier()

    stream_phase(s0, scale_w=False)
    next_phase(acc0_out)

    stream_phase(ne_s, scale_w=False)
    next_phase(acc2_out)

    stream_phase(e1, scale_w=True)
    next_phase(acc1_out)

    pltpu.sync_copy(zeros128, ra_v)

    def chunk_d(k, carry):
        load_idx(k, hqa_v, tqa_v, wqa_v, False, True)

        @plsc.parallel_loop(0, CHUNK, 1, unroll=4)
        def edge(e):
            ws = plsc.load_gather(wqa_v, [zeros16i, jnp.full((LANE,), e, jnp.int32)])
            ra_v[e, pl.ds(0, LANE)] = ws
        pltpu.sync_copy(ra_v, acc_sh.at[hqa_v.at[0]], add=True)
        return carry

    lax.fori_loop(0, NCHUNK, chunk_d, 0)
    plsc.subcore_barrier()
    _dump_acc(acc_sh, rs_out, cid, base)


def _prep_n_body(degp0, degp1, e0, dis, s0):
    deg = degp0[:, 0] + degp1[:, 0]
    d = jnp.where(deg > 0, lax.rsqrt(jnp.maximum(deg, 1e-30)), 0.0)
    dis[...] = d[:, None]
    s0[...] = d[:, None] * e0[...]


def _prep_n(degp0, degp1, e0):
    row = pl.BlockSpec((BN, D), lambda i: (i, 0))
    return pl.pallas_call(
        _prep_n_body,
        grid=(GRID_N,),
        in_specs=[row, row, row],
        out_specs=[pl.BlockSpec((BN, 1), lambda i: (i, 0)), row],
        out_shape=[
            jax.ShapeDtypeStruct((NP, 1), jnp.float32),
            jax.ShapeDtypeStruct((NP, D), jnp.float32),
        ],
    )(degp0, degp1, e0)


_EW = 128
_ER = E // _EW


def _prep_e_body(eps, b2, gum):
    lin = (2.0 * BIAS - 1.0) * eps[...] + (1.0 - BIAS)
    gum[...] = -jnp.log(-jnp.log(lin)) + b2[0, 0]


def _prep_e(eps_layer, b2_layer):
    out = pl.pallas_call(
        _prep_e_body,
        grid=(1,),
        in_specs=[
            pl.BlockSpec((_ER, _EW), lambda i: (0, 0)),
            pl.BlockSpec((1, 1), lambda i: (0, 0)),
        ],
        out_specs=pl.BlockSpec((_ER, _EW), lambda i: (0, 0)),
        out_shape=jax.ShapeDtypeStruct((_ER, _EW), jnp.float32),
    )(eps_layer.reshape(_ER, _EW), b2_layer.reshape(1, 1))
    return out.reshape(E)


def _dense_a_body(e1, e2, epsn, dis, w1a, w1b, b1e, w1n, b1n, w2n, b2n,
                  p1, p2, ne_s):
    p1[...] = jnp.dot(e1[...], w1a[...], preferred_element_type=jnp.float32) + b1e[...]
    p2[...] = jnp.dot(e1[...], w1b[...], preferred_element_type=jnp.float32)
    hid = jnp.maximum(jnp.dot(e2[...], w1n[...], preferred_element_type=jnp.float32) + b1n[...], 0.0)
    lg = jnp.dot(hid, w2n[...], preferred_element_type=jnp.float32) + b2n[...]
    lin = (2.0 * BIAS - 1.0) * epsn[...] + (1.0 - BIAS)
    gate = jax.nn.sigmoid(-jnp.log(-jnp.log(lin)) + lg)
    ne_s[...] = dis[...] * gate * e2[...]


def _dense_a(e1t, e2t, epsn, dis, w1a, w1b, b1e, w1n, b1n, w2n, b2n):
    row = pl.BlockSpec((BN, D), lambda i: (i, 0))
    mat = pl.BlockSpec((D, D), lambda i: (0, 0))
    vec = pl.BlockSpec((1, D), lambda i: (0, 0))
    return pl.pallas_call(
        _dense_a_body,
        grid=(GRID_N,),
        in_specs=[row, row, row, pl.BlockSpec((BN, 1), lambda i: (i, 0)),
                  mat, mat, vec, mat, vec, mat, vec],
        out_specs=[row, row, row],
        out_shape=[jax.ShapeDtypeStruct((NP, D), jnp.float32)] * 3,
    )(e1t, e2t, epsn, dis, w1a, w1b, b1e, w1n, b1n, w2n, b2n)


def _combine_body(e0, e1, e2, a00, a01, a10, a11, a20, a21, rs0, rs1, dis,
                  s0i, s1i, s2i,
                  e0n, e1n, e2n, s0n, s0o, s1o, s2o):
    rs = rs0[:, 0] + rs1[:, 0]
    inv = jnp.where(rs > 0, 1.0 / jnp.maximum(rs, 1e-30), 0.0)[:, None]
    d = dis[...]
    v0 = e0[...] + d * (a00[...] + a01[...])
    v1 = e1[...] + inv * (a10[...] + a11[...])
    v2 = e2[...] + d * (a20[...] + a21[...])
    e0n[...] = v0
    e1n[...] = v1
    e2n[...] = v2
    s0n[...] = d * v0
    s0o[...] = s0i[...] + v0
    s1o[...] = s1i[...] + v1
    s2o[...] = s2i[...] + v2


def _combine(e0t, e1t, e2t, a0p, a1p, a2p, rsp, dis, sums):
    row = pl.BlockSpec((BN, D), lambda i: (i, 0))
    return pl.pallas_call(
        _combine_body,
        grid=(GRID_N,),
        in_specs=[row, row, row, row, row, row, row, row, row,
                  row, row, pl.BlockSpec((BN, 1), lambda i: (i, 0)),
                  row, row, row],
        out_specs=[row] * 7,
        out_shape=[jax.ShapeDtypeStruct((NP, D), jnp.float32)] * 7,
    )(e0t, e1t, e2t, a0p[0], a0p[1], a1p[0], a1p[1], a2p[0], a2p[1],
      rsp[0], rsp[1], dis, sums[0], sums[1], sums[2])


def kernel(user_emb, item_emb, W1_e, b1_e, W2_e, b2_e, W1_n, b1_n, W2_n, b2_n,
           eps_edge, eps_node, all_h_list, all_t_list):
    e0 = jnp.pad(jnp.concatenate([user_emb, item_emb], axis=0),
                 ((0, NP - N), (0, 0)))
    h3 = jnp.pad(all_h_list.astype(jnp.int32), (0, EP - E),
                 constant_values=N).reshape(NW, NCHUNK, CHUNK)
    t3 = jnp.pad(all_t_list.astype(jnp.int32), (0, EP - E),
                 constant_values=N).reshape(NW, NCHUNK, CHUNK)
    ones128 = jnp.ones((CHUNK, D), jnp.float32)
    zeros128 = jnp.zeros((ZR, D), jnp.float32)

    degp = _deg_kernel(h3, ones128, zeros128)
    dis, s0t = _prep_n(degp[0], degp[1], e0)
    gum = jnp.stack([_prep_e(eps_edge[i, :, 0], b2_e[i]) for i in range(L_LAYERS)])

    e0t, e1t, e2t = e0, e0, e0
    sums = (e0, e0, e0)
    for i in range(L_LAYERS):
        epsn = jnp.pad(eps_node[i], ((0, NP - N), (0, 0)), constant_values=0.5)
        p1, p2, ne_s = _dense_a(
            e1t, e2t, epsn, dis,
            W1_e[i, :D, :], W1_e[i, D:, :], b1_e[i].reshape(1, D),
            W1_n[i], b1_n[i].reshape(1, D), W2_n[i], b2_n[i].reshape(1, D))
        gum_i = jnp.pad(gum[i], (0, EP - E)).reshape(NW, EPW)
        w2s = jnp.broadcast_to(W2_e[i, :, :1], (D, D))
        w_out = _edge_w_kernel(h3, t3, p1, p2, gum_i, w2s)
        acc0p, acc2p, acc1p, rsp = _scatter4_kernel(
            h3.reshape(NW, NCHUNK, 1, CHUNK), t3.reshape(NW, NCHUNK, 1, CHUNK),
            s0t, ne_s, e1t, w_out.reshape(NW, NCHUNK, 1, CHUNK), zeros128)
        e0t, e1t, e2t, s0t, su0, su1, su2 = _combine(
            e0t, e1t, e2t, acc0p, acc1p, acc2p, rsp, dis, sums)
        sums = (su0, su1, su2)

    return jnp.stack(sums, axis=0)[:, :N, :]
